# baseline (speedup 1.0000x reference)
import functools

import jax
import jax.numpy as jnp
from jax import lax
from jax.experimental import pallas as pl
from jax.experimental.pallas import tpu as pltpu
from jax.experimental.pallas import tpu_sc as plsc

S = 2048
E = 1024
N = 8
H = 4096
K = 2
P = S * K
MBLK = 128
PPAD = P + N * MBLK
NB = PPAD // MBLK
NBP = 48
TBLK = 256
NC = 2
NS = 16
NW = NC * NS
LANES = 16
SLOT_R = PPAD // NW
TOK_R = S // NW

_SC_MESH = plsc.VectorSubcoreMesh(core_axis_name="c", subcore_axis_name="s")
_SC_PARAMS = pltpu.CompilerParams(needs_layout_passes=False)


def _router_body(x_ref, rw1_ref, rb1_ref, rw2_ref, rb2_ref,
                 xbf_ref, wpair_ref, epair_ref, rank_ref, cnt_ref,
                 carry_ref):
    t = pl.program_id(0)
    xbf = x_ref[...].astype(jnp.bfloat16)
    xbf_ref[...] = xbf
    h = jnp.dot(xbf, rw1_ref[...].astype(jnp.bfloat16),
                preferred_element_type=jnp.float32) + rb1_ref[...]
    h = jnp.maximum(h, 0.0).astype(jnp.bfloat16)
    s = jnp.dot(h, rw2_ref[...].astype(jnp.bfloat16),
                preferred_element_type=jnp.float32) + rb2_ref[...]
    lane = lax.broadcasted_iota(jnp.int32, s.shape, 1)
    m1 = jnp.max(s, axis=1, keepdims=True)
    a1 = jnp.min(jnp.where(s == m1, lane, N), axis=1, keepdims=True)
    sm = jnp.where(lane == a1, -jnp.inf, s)
    m2 = jnp.max(sm, axis=1, keepdims=True)
    a2 = jnp.min(jnp.where(sm == m2, lane, N), axis=1, keepdims=True)
    e2 = jnp.exp(m2 - m1)
    w1 = 1.0 / (1.0 + e2)
    w2 = e2 / (1.0 + e2)
    wpair_ref[...] = jnp.concatenate([w1, w2], axis=1)
    epair_ref[...] = jnp.concatenate([a1, a2], axis=1)

    oh = (lane == a1).astype(jnp.float32) + (lane == a2).astype(jnp.float32)
    r_i = lax.broadcasted_iota(jnp.int32, (TBLK, TBLK), 0)
    c_i = lax.broadcasted_iota(jnp.int32, (TBLK, TBLK), 1)
    tri = (c_i < r_i).astype(jnp.bfloat16)
    cum = jnp.dot(tri, oh.astype(jnp.bfloat16),
                  preferred_element_type=jnp.float32)

    @pl.when(t == 0)
    def _():
        carry_ref[...] = jnp.zeros_like(carry_ref)

    carry = carry_ref[...]
    cumg = cum + carry
    rank1 = jnp.sum(jnp.where(lane == a1, cumg, 0.0), axis=1, keepdims=True)
    rank2 = jnp.sum(jnp.where(lane == a2, cumg, 0.0), axis=1, keepdims=True)
    rank_ref[...] = jnp.concatenate([rank1, rank2], axis=1).astype(jnp.int32)
    newc = carry + jnp.sum(oh, axis=0, keepdims=True)
    carry_ref[...] = newc
    cnt_ref[...] = jnp.concatenate(
        [newc, jnp.zeros((1, 16 - N), jnp.float32)], axis=1).astype(jnp.int32)


def _route_sc_body(ep_hbm, rk_hbm, w_hbm, cnt_hbm,
                   gidx_hbm, wsort_hbm, pos_hbm, bexp_hbm,
                   ep_v, rk_v, w_v, cnt_v, offs_v, offsi_v,
                   gloc_v, wloc_v, pos_v, bexp_v, sem):
    cid = lax.axis_index("c")
    sid = lax.axis_index("s")
    wid = sid * NC + cid

    pltpu.sync_copy(ep_hbm, ep_v)
    pltpu.sync_copy(rk_hbm, rk_v)
    pltpu.sync_copy(w_hbm, w_v)
    pltpu.sync_copy(cnt_hbm.at[0], cnt_v)

    cnt = cnt_v[...]
    padded = ((cnt + (MBLK - 1)) // MBLK) * MBLK
    offsi = plsc.cumsum(padded)
    offs = offsi - padded
    offs_v[...] = offs
    offsi_v[...] = offsi

    pbase = wid * (P // NW)

    @pl.loop(0, P // NW, step=LANES)
    def _(j):
        e = ep_v[pl.ds(pbase + j, LANES)]
        r = rk_v[pl.ds(pbase + j, LANES)]
        pos_v[pl.ds(j, LANES)] = plsc.load_gather(offs_v, [e]) + r

    pltpu.sync_copy(pos_v, pos_hbm.at[pl.ds(pbase, P // NW)])

    lo = wid * SLOT_R

    @pl.loop(0, SLOT_R, step=LANES)
    def _(j):
        gloc_v[pl.ds(j, LANES)] = jnp.zeros((LANES,), jnp.int32)
        wloc_v[pl.ds(j, LANES)] = jnp.zeros((LANES,), jnp.float32)

    @pl.loop(0, P, step=LANES)
    def _(i):
        e = ep_v[pl.ds(i, LANES)]
        r = rk_v[pl.ds(i, LANES)]
        wv = w_v[pl.ds(i, LANES)]
        pos = plsc.load_gather(offs_v, [e]) + r
        m = (pos >= lo) & (pos < lo + SLOT_R)
        il = jnp.where(m, pos - lo, 0)
        tok = (i + lax.iota(jnp.int32, LANES)) // K
        plsc.store_scatter(gloc_v, [il], tok, mask=m)
        plsc.store_scatter(wloc_v, [il], wv, mask=m)

    pltpu.sync_copy(gloc_v, gidx_hbm.at[pl.ds(lo, SLOT_R)])
    pltpu.sync_copy(wloc_v, wsort_hbm.at[pl.ds(lo, SLOT_R)])

    @pl.when(wid == 0)
    def _():
        lane16 = lax.iota(jnp.int32, LANES)

        @pl.loop(0, NBP, step=LANES)
        def _(j):
            row = (j + lane16) * MBLK
            acc = jnp.zeros((LANES,), jnp.int32)
            for e in range(N):
                off_e = jnp.sum(jnp.where(lane16 == e, offsi_v[...], 0))
                acc += jnp.where(row >= off_e, 1, 0)
            bexp_v[pl.ds(j, LANES)] = jnp.minimum(acc, N - 1)

        pltpu.sync_copy(bexp_v, bexp_hbm)


def _gather_sc_body(xbf_hbm, gidx_hbm, xs_hbm, idx_v, rows_v, sem):
    cid = lax.axis_index("c")
    sid = lax.axis_index("s")
    wid = sid * NC + cid
    lo = wid * SLOT_R
    pltpu.sync_copy(gidx_hbm.at[pl.ds(lo, SLOT_R)], idx_v)
    half = SLOT_R // 2
    cp1 = pltpu.async_copy(xbf_hbm.at[idx_v.at[pl.ds(0, half)]],
                           rows_v.at[pl.ds(0, half)], sem)
    cp2 = pltpu.async_copy(xbf_hbm.at[idx_v.at[pl.ds(half, half)]],
                           rows_v.at[pl.ds(half, half)], sem)
    cp1.wait()
    cp2.wait()
    pltpu.sync_copy(rows_v, xs_hbm.at[pl.ds(lo, SLOT_R)])


def _gmm_body(bexp_ref, xs_ref, ew1_ref, eb1_ref, ew2_ref, eb2_ref, w_ref,
              o_ref):
    x = xs_ref[...]
    h = jnp.dot(x, ew1_ref[0], preferred_element_type=jnp.float32) + eb1_ref[0]
    h = jnp.maximum(h, 0.0).astype(jnp.bfloat16)
    y = jnp.dot(h, ew2_ref[0], preferred_element_type=jnp.float32) + eb2_ref[0]
    o_ref[...] = y * w_ref[...]


def _combine_sc_body(ys_hbm, pos_hbm, out_hbm, idx_v, rows_v, out_v, sem):
    cid = lax.axis_index("c")
    sid = lax.axis_index("s")
    wid = sid * NC + cid
    tbase = wid * TOK_R
    pltpu.sync_copy(pos_hbm.at[pl.ds(tbase * K, TOK_R * K)], idx_v)
    rows_half = TOK_R * K // 2
    toks_half = TOK_R // 2
    for c in range(2):
        pltpu.async_copy(ys_hbm.at[idx_v.at[pl.ds(c * rows_half, rows_half)]],
                         rows_v, sem).wait()

        @pl.loop(0, toks_half)
        def _(u):
            @pl.loop(0, E, step=LANES)
            def _(j):
                a = rows_v[2 * u, pl.ds(j, LANES)]
                b = rows_v[2 * u + 1, pl.ds(j, LANES)]
                out_v[u, pl.ds(j, LANES)] = a + b

        pltpu.sync_copy(out_v, out_hbm.at[pl.ds(tbase + c * toks_half,
                                                toks_half)])


@jax.jit
def kernel(inputs, rw1, rb1, rw2, rb2, ew1, eb1, ew2, eb2):
    x2 = inputs.reshape(S, E)

    xbf, wpair, epair, rank, cnt = pl.pallas_call(
        _router_body,
        grid=(S // TBLK,),
        in_specs=[
            pl.BlockSpec((TBLK, E), lambda i: (i, 0)),
            pl.BlockSpec((E, E), lambda i: (0, 0)),
            pl.BlockSpec((E,), lambda i: (0,)),
            pl.BlockSpec((E, N), lambda i: (0, 0)),
            pl.BlockSpec((N,), lambda i: (0,)),
        ],
        out_specs=[
            pl.BlockSpec((TBLK, E), lambda i: (i, 0)),
            pl.BlockSpec((TBLK, K), lambda i: (i, 0)),
            pl.BlockSpec((TBLK, K), lambda i: (i, 0)),
            pl.BlockSpec((TBLK, K), lambda i: (i, 0)),
            pl.BlockSpec((1, 16), lambda i: (0, 0)),
        ],
        out_shape=[
            jax.ShapeDtypeStruct((S, E), jnp.bfloat16),
            jax.ShapeDtypeStruct((S, K), jnp.float32),
            jax.ShapeDtypeStruct((S, K), jnp.int32),
            jax.ShapeDtypeStruct((S, K), jnp.int32),
            jax.ShapeDtypeStruct((1, 16), jnp.int32),
        ],
        scratch_shapes=[pltpu.VMEM((1, N), jnp.float32)],
    )(x2, rw1, rb1, rw2, rb2)

    route_sc = pl.kernel(
        _route_sc_body,
        out_type=[
            jax.ShapeDtypeStruct((PPAD,), jnp.int32),
            jax.ShapeDtypeStruct((PPAD,), jnp.float32),
            jax.ShapeDtypeStruct((P,), jnp.int32),
            jax.ShapeDtypeStruct((NBP,), jnp.int32),
        ],
        mesh=_SC_MESH,
        scratch_types=[
            pltpu.VMEM((P,), jnp.int32),
            pltpu.VMEM((P,), jnp.int32),
            pltpu.VMEM((P,), jnp.float32),
            pltpu.VMEM((16,), jnp.int32),
            pltpu.VMEM((16,), jnp.int32),
            pltpu.VMEM((16,), jnp.int32),
            pltpu.VMEM((SLOT_R,), jnp.int32),
            pltpu.VMEM((SLOT_R,), jnp.float32),
            pltpu.VMEM((P // NW,), jnp.int32),
            pltpu.VMEM((NBP,), jnp.int32),
            pltpu.SemaphoreType.DMA,
        ],
        compiler_params=_SC_PARAMS,
    )
    gidx, wsort, pos, bexp = route_sc(
        epair.reshape(P), rank.reshape(P), wpair.reshape(P), cnt)

    EW = E // 2
    xbf_i32 = lax.bitcast_convert_type(
        xbf.reshape(S, EW, 2), jnp.int32).reshape(S, EW // 128, 128)
    gather_sc = pl.kernel(
        _gather_sc_body,
        out_type=jax.ShapeDtypeStruct((PPAD, EW // 128, 128), jnp.int32),
        mesh=_SC_MESH,
        scratch_types=[
            pltpu.VMEM((SLOT_R,), jnp.int32),
            pltpu.VMEM((SLOT_R, EW // 128, 128), jnp.int32),
            pltpu.SemaphoreType.DMA,
        ],
        compiler_params=_SC_PARAMS,
    )
    xs_i32 = gather_sc(xbf_i32, gidx)
    xs = lax.bitcast_convert_type(
        xs_i32.reshape(PPAD, EW), jnp.bfloat16).reshape(PPAD, E)

    grid_spec = pltpu.PrefetchScalarGridSpec(
        num_scalar_prefetch=1,
        grid=(NB,),
        in_specs=[
            pl.BlockSpec((MBLK, E), lambda b, s: (b, 0)),
            pl.BlockSpec((1, E, H), lambda b, s: (s[b], 0, 0)),
            pl.BlockSpec((1, 1, H), lambda b, s: (s[b], 0, 0)),
            pl.BlockSpec((1, H, E), lambda b, s: (s[b], 0, 0)),
            pl.BlockSpec((1, 1, E), lambda b, s: (s[b], 0, 0)),
            pl.BlockSpec((MBLK, 1), lambda b, s: (b, 0)),
        ],
        out_specs=pl.BlockSpec((MBLK, E), lambda b, s: (b, 0)),
    )
    ys = pl.pallas_call(
        _gmm_body,
        grid_spec=grid_spec,
        out_shape=jax.ShapeDtypeStruct((PPAD, E), jnp.float32),
        compiler_params=pltpu.CompilerParams(
            dimension_semantics=("arbitrary",),
        ),
    )(bexp, xs, ew1.astype(jnp.bfloat16),
      eb1.reshape(N, 1, H), ew2.astype(jnp.bfloat16), eb2.reshape(N, 1, E),
      wsort.reshape(PPAD, 1))

    combine_sc = pl.kernel(
        _combine_sc_body,
        out_type=jax.ShapeDtypeStruct((S, E), jnp.float32),
        mesh=_SC_MESH,
        scratch_types=[
            pltpu.VMEM((TOK_R * K,), jnp.int32),
            pltpu.VMEM((TOK_R * K // 2, E), jnp.float32),
            pltpu.VMEM((TOK_R // 2, E), jnp.float32),
            pltpu.SemaphoreType.DMA,
        ],
        compiler_params=_SC_PARAMS,
    )
    out = combine_sc(ys, pos)
    return out.reshape(1, S, E)


# speedup vs baseline: 1.3111x; 1.3111x over previous
import functools

import jax
import jax.numpy as jnp
from jax import lax
from jax.experimental import pallas as pl
from jax.experimental.pallas import tpu as pltpu
from jax.experimental.pallas import tpu_sc as plsc

S = 2048
E = 1024
N = 8
H = 4096
K = 2
P = S * K
MBLK = 128
PPAD = P + N * MBLK
NB = PPAD // MBLK
NBP = 48
TBLK = 256
NC = 2
NS = 16
NW = NC * NS
LANES = 16
SLOT_R = PPAD // NW
TOK_R = S // NW

_SC_MESH = plsc.VectorSubcoreMesh(core_axis_name="c", subcore_axis_name="s")
_SC_PARAMS = pltpu.CompilerParams(needs_layout_passes=False)


def _router_body(x_ref, rw1_ref, rb1_ref, rw2_ref, rb2_ref,
                 wpair_ref, epair_ref, rank_ref, cnt_ref,
                 carry_ref):
    t = pl.program_id(0)
    xbf = x_ref[...].astype(jnp.bfloat16)
    h = jnp.dot(xbf, rw1_ref[...].astype(jnp.bfloat16),
                preferred_element_type=jnp.float32) + rb1_ref[...]
    h = jnp.maximum(h, 0.0).astype(jnp.bfloat16)
    s = jnp.dot(h, rw2_ref[...].astype(jnp.bfloat16),
                preferred_element_type=jnp.float32) + rb2_ref[...]
    lane = lax.broadcasted_iota(jnp.int32, s.shape, 1)
    m1 = jnp.max(s, axis=1, keepdims=True)
    a1 = jnp.min(jnp.where(s == m1, lane, N), axis=1, keepdims=True)
    sm = jnp.where(lane == a1, -jnp.inf, s)
    m2 = jnp.max(sm, axis=1, keepdims=True)
    a2 = jnp.min(jnp.where(sm == m2, lane, N), axis=1, keepdims=True)
    e2 = jnp.exp(m2 - m1)
    w1 = 1.0 / (1.0 + e2)
    w2 = e2 / (1.0 + e2)
    wpair_ref[...] = jnp.concatenate([w1, w2], axis=1)
    epair_ref[...] = jnp.concatenate([a1, a2], axis=1)

    oh = (lane == a1).astype(jnp.float32) + (lane == a2).astype(jnp.float32)
    r_i = lax.broadcasted_iota(jnp.int32, (TBLK, TBLK), 0)
    c_i = lax.broadcasted_iota(jnp.int32, (TBLK, TBLK), 1)
    tri = (c_i < r_i).astype(jnp.bfloat16)
    cum = jnp.dot(tri, oh.astype(jnp.bfloat16),
                  preferred_element_type=jnp.float32)

    @pl.when(t == 0)
    def _():
        carry_ref[...] = jnp.zeros_like(carry_ref)

    carry = carry_ref[...]
    cumg = cum + carry
    rank1 = jnp.sum(jnp.where(lane == a1, cumg, 0.0), axis=1, keepdims=True)
    rank2 = jnp.sum(jnp.where(lane == a2, cumg, 0.0), axis=1, keepdims=True)
    rank_ref[...] = jnp.concatenate([rank1, rank2], axis=1).astype(jnp.int32)
    newc = carry + jnp.sum(oh, axis=0, keepdims=True)
    carry_ref[...] = newc
    cnt_ref[...] = jnp.concatenate(
        [newc, jnp.zeros((1, 16 - N), jnp.float32)], axis=1).astype(jnp.int32)


def _route_sc_body(ep_hbm, rk_hbm, w_hbm, cnt_hbm,
                   gidx_hbm, wsort_hbm, pos_hbm, bexp_hbm,
                   ep_v, rk_v, w_v, cnt_v, offs_v, offsi_v,
                   gloc_v, wloc_v, pos_v, bexp_v, sem):
    cid = lax.axis_index("c")
    sid = lax.axis_index("s")
    wid = sid * NC + cid

    pltpu.sync_copy(ep_hbm, ep_v)
    pltpu.sync_copy(rk_hbm, rk_v)
    pltpu.sync_copy(w_hbm, w_v)
    pltpu.sync_copy(cnt_hbm.at[0], cnt_v)

    cnt = cnt_v[...]
    padded = ((cnt + (MBLK - 1)) // MBLK) * MBLK
    offsi = plsc.cumsum(padded)
    offs = offsi - padded
    offs_v[...] = offs
    offsi_v[...] = offsi

    pbase = wid * (P // NW)

    @pl.loop(0, P // NW, step=LANES)
    def _(j):
        e = ep_v[pl.ds(pbase + j, LANES)]
        r = rk_v[pl.ds(pbase + j, LANES)]
        pos_v[pl.ds(j, LANES)] = plsc.load_gather(offs_v, [e]) + r

    pltpu.sync_copy(pos_v, pos_hbm.at[pl.ds(pbase, P // NW)])

    lo = wid * SLOT_R

    @pl.loop(0, SLOT_R, step=LANES)
    def _(j):
        gloc_v[pl.ds(j, LANES)] = jnp.zeros((LANES,), jnp.int32)
        wloc_v[pl.ds(j, LANES)] = jnp.zeros((LANES,), jnp.float32)

    @pl.loop(0, P, step=LANES)
    def _(i):
        e = ep_v[pl.ds(i, LANES)]
        r = rk_v[pl.ds(i, LANES)]
        wv = w_v[pl.ds(i, LANES)]
        pos = plsc.load_gather(offs_v, [e]) + r
        m = (pos >= lo) & (pos < lo + SLOT_R)
        il = jnp.where(m, pos - lo, 0)
        tok = (i + lax.iota(jnp.int32, LANES)) // K
        plsc.store_scatter(gloc_v, [il], tok, mask=m)
        plsc.store_scatter(wloc_v, [il], wv, mask=m)

    pltpu.sync_copy(gloc_v, gidx_hbm.at[pl.ds(lo, SLOT_R)])
    pltpu.sync_copy(wloc_v, wsort_hbm.at[pl.ds(lo, SLOT_R)])

    @pl.when(wid == 0)
    def _():
        lane16 = lax.iota(jnp.int32, LANES)

        @pl.loop(0, NBP, step=LANES)
        def _(j):
            row = (j + lane16) * MBLK
            acc = jnp.zeros((LANES,), jnp.int32)
            for e in range(N):
                off_e = jnp.sum(jnp.where(lane16 == e, offsi_v[...], 0))
                acc += jnp.where(row >= off_e, 1, 0)
            bexp_v[pl.ds(j, LANES)] = jnp.minimum(acc, N - 1)

        pltpu.sync_copy(bexp_v, bexp_hbm)


GCH = 40


def _gather_sc_body(x_hbm, gidx_hbm, xs_hbm, idx_v, rows_a, rows_b, sem_a,
                    sem_b, sem_w):
    cid = lax.axis_index("c")
    sid = lax.axis_index("s")
    wid = sid * NC + cid
    lo = wid * SLOT_R
    pltpu.sync_copy(gidx_hbm.at[pl.ds(lo, SLOT_R)], idx_v)
    bufs = (rows_a, rows_b)
    sems = (sem_a, sem_b)
    cps = [None, None, None, None]
    cps[0] = pltpu.async_copy(x_hbm.at[idx_v.at[pl.ds(0, GCH)]], rows_a,
                              sem_a)
    cps[1] = pltpu.async_copy(x_hbm.at[idx_v.at[pl.ds(GCH, GCH)]], rows_b,
                              sem_b)
    for c in range(4):
        cps[c].wait()
        wr = pltpu.async_copy(bufs[c % 2],
                              xs_hbm.at[pl.ds(lo + c * GCH, GCH)], sem_w)
        wr.wait()
        if c + 2 < 4:
            cps[c + 2] = pltpu.async_copy(
                x_hbm.at[idx_v.at[pl.ds((c + 2) * GCH, GCH)]], bufs[c % 2],
                sem_a if c % 2 == 0 else sem_b)


def _gmm_body(bexp_ref, xs_ref, ew1_ref, eb1_ref, ew2_ref, eb2_ref, w_ref,
              o_ref):
    x = xs_ref[...].astype(jnp.bfloat16)
    h = jnp.dot(x, ew1_ref[0], preferred_element_type=jnp.float32) + eb1_ref[0]
    h = jnp.maximum(h, 0.0).astype(jnp.bfloat16)
    y = jnp.dot(h, ew2_ref[0], preferred_element_type=jnp.float32) + eb2_ref[0]
    o_ref[...] = y * w_ref[...]


def _combine_sc_body(ys_hbm, pos_hbm, out_hbm, idx_v, rows_v, out_v, sem):
    cid = lax.axis_index("c")
    sid = lax.axis_index("s")
    wid = sid * NC + cid
    tbase = wid * TOK_R
    pltpu.sync_copy(pos_hbm.at[pl.ds(tbase * K, TOK_R * K)], idx_v)
    rows_half = TOK_R * K // 2
    toks_half = TOK_R // 2
    for c in range(2):
        pltpu.async_copy(ys_hbm.at[idx_v.at[pl.ds(c * rows_half, rows_half)]],
                         rows_v, sem).wait()

        @pl.loop(0, toks_half)
        def _(u):
            @pl.loop(0, E, step=LANES)
            def _(j):
                a = rows_v[2 * u, pl.ds(j, LANES)]
                b = rows_v[2 * u + 1, pl.ds(j, LANES)]
                out_v[u, pl.ds(j, LANES)] = a + b

        pltpu.sync_copy(out_v, out_hbm.at[pl.ds(tbase + c * toks_half,
                                                toks_half)])


@jax.jit
def kernel(inputs, rw1, rb1, rw2, rb2, ew1, eb1, ew2, eb2):
    x2 = inputs.reshape(S, E)

    wpair, epair, rank, cnt = pl.pallas_call(
        _router_body,
        grid=(S // TBLK,),
        in_specs=[
            pl.BlockSpec((TBLK, E), lambda i: (i, 0)),
            pl.BlockSpec((E, E), lambda i: (0, 0)),
            pl.BlockSpec((E,), lambda i: (0,)),
            pl.BlockSpec((E, N), lambda i: (0, 0)),
            pl.BlockSpec((N,), lambda i: (0,)),
        ],
        out_specs=[
            pl.BlockSpec((TBLK, K), lambda i: (i, 0)),
            pl.BlockSpec((TBLK, K), lambda i: (i, 0)),
            pl.BlockSpec((TBLK, K), lambda i: (i, 0)),
            pl.BlockSpec((1, 16), lambda i: (0, 0)),
        ],
        out_shape=[
            jax.ShapeDtypeStruct((S, K), jnp.float32),
            jax.ShapeDtypeStruct((S, K), jnp.int32),
            jax.ShapeDtypeStruct((S, K), jnp.int32),
            jax.ShapeDtypeStruct((1, 16), jnp.int32),
        ],
        scratch_shapes=[pltpu.VMEM((1, N), jnp.float32)],
    )(x2, rw1, rb1, rw2, rb2)

    route_sc = pl.kernel(
        _route_sc_body,
        out_type=[
            jax.ShapeDtypeStruct((PPAD,), jnp.int32),
            jax.ShapeDtypeStruct((PPAD,), jnp.float32),
            jax.ShapeDtypeStruct((P,), jnp.int32),
            jax.ShapeDtypeStruct((NBP,), jnp.int32),
        ],
        mesh=_SC_MESH,
        scratch_types=[
            pltpu.VMEM((P,), jnp.int32),
            pltpu.VMEM((P,), jnp.int32),
            pltpu.VMEM((P,), jnp.float32),
            pltpu.VMEM((16,), jnp.int32),
            pltpu.VMEM((16,), jnp.int32),
            pltpu.VMEM((16,), jnp.int32),
            pltpu.VMEM((SLOT_R,), jnp.int32),
            pltpu.VMEM((SLOT_R,), jnp.float32),
            pltpu.VMEM((P // NW,), jnp.int32),
            pltpu.VMEM((NBP,), jnp.int32),
            pltpu.SemaphoreType.DMA,
        ],
        compiler_params=_SC_PARAMS,
    )
    gidx, wsort, pos, bexp = route_sc(
        epair.reshape(P), rank.reshape(P), wpair.reshape(P), cnt)

    gather_sc = pl.kernel(
        _gather_sc_body,
        out_type=jax.ShapeDtypeStruct((PPAD, E), jnp.float32),
        mesh=_SC_MESH,
        scratch_types=[
            pltpu.VMEM((SLOT_R,), jnp.int32),
            pltpu.VMEM((GCH, E), jnp.float32),
            pltpu.VMEM((GCH, E), jnp.float32),
            pltpu.SemaphoreType.DMA,
            pltpu.SemaphoreType.DMA,
            pltpu.SemaphoreType.DMA,
        ],
        compiler_params=_SC_PARAMS,
    )
    xs = gather_sc(x2, gidx)

    grid_spec = pltpu.PrefetchScalarGridSpec(
        num_scalar_prefetch=1,
        grid=(NB,),
        in_specs=[
            pl.BlockSpec((MBLK, E), lambda b, s: (b, 0)),
            pl.BlockSpec((1, E, H), lambda b, s: (s[b], 0, 0)),
            pl.BlockSpec((1, 1, H), lambda b, s: (s[b], 0, 0)),
            pl.BlockSpec((1, H, E), lambda b, s: (s[b], 0, 0)),
            pl.BlockSpec((1, 1, E), lambda b, s: (s[b], 0, 0)),
            pl.BlockSpec((MBLK, 1), lambda b, s: (b, 0)),
        ],
        out_specs=pl.BlockSpec((MBLK, E), lambda b, s: (b, 0)),
    )
    ys = pl.pallas_call(
        _gmm_body,
        grid_spec=grid_spec,
        out_shape=jax.ShapeDtypeStruct((PPAD, E), jnp.float32),
        compiler_params=pltpu.CompilerParams(
            dimension_semantics=("arbitrary",),
        ),
    )(bexp, xs, ew1.astype(jnp.bfloat16),
      eb1.reshape(N, 1, H), ew2.astype(jnp.bfloat16), eb2.reshape(N, 1, E),
      wsort.reshape(PPAD, 1))

    combine_sc = pl.kernel(
        _combine_sc_body,
        out_type=jax.ShapeDtypeStruct((S, E), jnp.float32),
        mesh=_SC_MESH,
        scratch_types=[
            pltpu.VMEM((TOK_R * K,), jnp.int32),
            pltpu.VMEM((TOK_R * K // 2, E), jnp.float32),
            pltpu.VMEM((TOK_R // 2, E), jnp.float32),
            pltpu.SemaphoreType.DMA,
        ],
        compiler_params=_SC_PARAMS,
    )
    out = combine_sc(ys, pos)
    return out.reshape(1, S, E)


# speedup vs baseline: 1.5302x; 1.1671x over previous
import functools

import jax
import jax.numpy as jnp
from jax import lax
from jax.experimental import pallas as pl
from jax.experimental.pallas import tpu as pltpu
from jax.experimental.pallas import tpu_sc as plsc

S = 2048
E = 1024
N = 8
H = 4096
K = 2
P = S * K
MBLK = 128
PPAD = P + N * MBLK
NB = PPAD // MBLK
TBLK = 256
NC = 2
NS = 16
NW = NC * NS
LANES = 16
TOK_R = S // NW

_SC_MESH = plsc.VectorSubcoreMesh(core_axis_name="c", subcore_axis_name="s")
_SC_PARAMS = pltpu.CompilerParams(needs_layout_passes=False)


def _router_body(x_ref, rw1_ref, rb1_ref, rw2_ref, rb2_ref,
                 wpair_ref, pos_ref, bexp_ref,
                 ep_s, rk_s, carry_ref):
    t = pl.program_id(0)
    nblocks = pl.num_programs(0)
    xbf = x_ref[...].astype(jnp.bfloat16)
    h = jnp.dot(xbf, rw1_ref[...].astype(jnp.bfloat16),
                preferred_element_type=jnp.float32) + rb1_ref[...]
    h = jnp.maximum(h, 0.0).astype(jnp.bfloat16)
    s = jnp.dot(h, rw2_ref[...].astype(jnp.bfloat16),
                preferred_element_type=jnp.float32) + rb2_ref[...]
    lane = lax.broadcasted_iota(jnp.int32, s.shape, 1)
    m1 = jnp.max(s, axis=1, keepdims=True)
    a1 = jnp.min(jnp.where(s == m1, lane, N), axis=1, keepdims=True)
    sm = jnp.where(lane == a1, -jnp.inf, s)
    m2 = jnp.max(sm, axis=1, keepdims=True)
    a2 = jnp.min(jnp.where(sm == m2, lane, N), axis=1, keepdims=True)
    e2 = jnp.exp(m2 - m1)
    w1 = 1.0 / (1.0 + e2)
    w2 = e2 / (1.0 + e2)
    wpair_ref[...] = jnp.concatenate([w1, w2], axis=1)

    oh = (lane == a1).astype(jnp.float32) + (lane == a2).astype(jnp.float32)
    r_i = lax.broadcasted_iota(jnp.int32, (TBLK, TBLK), 0)
    c_i = lax.broadcasted_iota(jnp.int32, (TBLK, TBLK), 1)
    tri = (c_i < r_i).astype(jnp.bfloat16)
    cum = jnp.dot(tri, oh.astype(jnp.bfloat16),
                  preferred_element_type=jnp.float32)

    @pl.when(t == 0)
    def _():
        carry_ref[...] = jnp.zeros_like(carry_ref)

    carry = carry_ref[...]
    cumg = cum + carry
    rank1 = jnp.sum(jnp.where(lane == a1, cumg, 0.0), axis=1, keepdims=True)
    rank2 = jnp.sum(jnp.where(lane == a2, cumg, 0.0), axis=1, keepdims=True)
    ep_s[pl.ds(t * TBLK, TBLK), :] = jnp.concatenate([a1, a2], axis=1)
    rk_s[pl.ds(t * TBLK, TBLK), :] = jnp.concatenate(
        [rank1, rank2], axis=1).astype(jnp.int32)
    carry_ref[...] = carry + jnp.sum(oh, axis=0, keepdims=True)

    @pl.when(t == nblocks - 1)
    def _():
        cnt = carry_ref[...]
        padded = jnp.floor((cnt + (MBLK - 1)) / MBLK) * MBLK
        ui = lax.broadcasted_iota(jnp.int32, (N, N), 0)
        uj = lax.broadcasted_iota(jnp.int32, (N, N), 1)
        offs = jnp.dot(padded.astype(jnp.bfloat16),
                       (ui < uj).astype(jnp.bfloat16),
                       preferred_element_type=jnp.float32)
        offsi = offs + padded
        lane8 = lax.broadcasted_iota(jnp.int32, (1, N), 1)
        ep = ep_s[...]
        rk = rk_s[...].astype(jnp.float32)
        posf = rk
        for e in range(N):
            off_e = jnp.sum(jnp.where(lane8 == e, offs, 0.0))
            posf = posf + jnp.where(ep == e, off_e, 0.0)
        pos_ref[...] = posf.astype(jnp.int32)
        lane128 = lax.broadcasted_iota(jnp.int32, (1, 128), 1)
        row = (lane128 * MBLK).astype(jnp.float32)
        acc = jnp.zeros((1, 128), jnp.int32)
        for e in range(N):
            offi_e = jnp.sum(jnp.where(lane8 == e, offsi, 0.0))
            acc = acc + jnp.where(row >= offi_e, 1, 0)
        bexp_ref[...] = jnp.minimum(acc, N - 1)


def _gmm_body(bexp_ref, posT_ref, wT_ref, xb_ref,
              ew1_ref, eb1_ref, ew2_ref, eb2_ref, o_ref):
    b = pl.program_id(0)
    slot = lax.broadcasted_iota(jnp.int32, (MBLK, S), 0) + b * MBLK
    p0 = posT_ref[0:1, :]
    p1 = posT_ref[1:2, :]
    sel0 = slot == p0
    sel1 = slot == p1
    perm = (sel0 | sel1).astype(jnp.bfloat16)
    xs = jnp.dot(perm, xb_ref[...],
                 preferred_element_type=jnp.float32).astype(jnp.bfloat16)
    ws = jnp.sum(jnp.where(sel0, wT_ref[0:1, :], 0.0) +
                 jnp.where(sel1, wT_ref[1:2, :], 0.0),
                 axis=1, keepdims=True)
    h = jnp.dot(xs, ew1_ref[0], preferred_element_type=jnp.float32) + eb1_ref[0]
    h = jnp.maximum(h, 0.0).astype(jnp.bfloat16)
    y = jnp.dot(h, ew2_ref[0], preferred_element_type=jnp.float32) + eb2_ref[0]
    o_ref[...] = y * ws


def _combine_sc_body(ys_hbm, posT_hbm, out_hbm, idx0_v, idx1_v, rows_a,
                     rows_b, sem):
    cid = lax.axis_index("c")
    sid = lax.axis_index("s")
    wid = sid * NC + cid
    tbase = wid * TOK_R
    pltpu.sync_copy(posT_hbm.at[0, pl.ds(tbase, TOK_R)], idx0_v)
    pltpu.sync_copy(posT_hbm.at[1, pl.ds(tbase, TOK_R)], idx1_v)
    half = TOK_R // 2
    for c in range(2):
        cp1 = pltpu.async_copy(
            ys_hbm.at[idx0_v.at[pl.ds(c * half, half)]], rows_a, sem)
        cp2 = pltpu.async_copy(
            ys_hbm.at[idx1_v.at[pl.ds(c * half, half)]], rows_b, sem)
        cp1.wait()
        cp2.wait()

        @pl.loop(0, half)
        def _(u):
            @pl.loop(0, E, step=LANES)
            def _(j):
                rows_a[u, pl.ds(j, LANES)] = (rows_a[u, pl.ds(j, LANES)] +
                                              rows_b[u, pl.ds(j, LANES)])

        pltpu.sync_copy(rows_a, out_hbm.at[pl.ds(tbase + c * half, half)])


@jax.jit
def kernel(inputs, rw1, rb1, rw2, rb2, ew1, eb1, ew2, eb2):
    x2 = inputs.reshape(S, E)

    wpair, pos, bexp = pl.pallas_call(
        _router_body,
        grid=(S // TBLK,),
        in_specs=[
            pl.BlockSpec((TBLK, E), lambda i: (i, 0)),
            pl.BlockSpec((E, E), lambda i: (0, 0)),
            pl.BlockSpec((E,), lambda i: (0,)),
            pl.BlockSpec((E, N), lambda i: (0, 0)),
            pl.BlockSpec((N,), lambda i: (0,)),
        ],
        out_specs=[
            pl.BlockSpec((TBLK, K), lambda i: (i, 0)),
            pl.BlockSpec((S, K), lambda i: (0, 0)),
            pl.BlockSpec((1, 128), lambda i: (0, 0)),
        ],
        out_shape=[
            jax.ShapeDtypeStruct((S, K), jnp.float32),
            jax.ShapeDtypeStruct((S, K), jnp.int32),
            jax.ShapeDtypeStruct((1, 128), jnp.int32),
        ],
        scratch_shapes=[
            pltpu.VMEM((S, K), jnp.int32),
            pltpu.VMEM((S, K), jnp.int32),
            pltpu.VMEM((1, N), jnp.float32),
        ],
    )(x2, rw1, rb1, rw2, rb2)

    posT = pos.T
    wT = wpair.T
    xb = x2.astype(jnp.bfloat16)

    grid_spec = pltpu.PrefetchScalarGridSpec(
        num_scalar_prefetch=1,
        grid=(NB,),
        in_specs=[
            pl.BlockSpec((K, S), lambda b, s: (0, 0)),
            pl.BlockSpec((K, S), lambda b, s: (0, 0)),
            pl.BlockSpec((S, E), lambda b, s: (0, 0)),
            pl.BlockSpec((1, E, H), lambda b, s: (s[b], 0, 0)),
            pl.BlockSpec((1, 1, H), lambda b, s: (s[b], 0, 0)),
            pl.BlockSpec((1, H, E), lambda b, s: (s[b], 0, 0)),
            pl.BlockSpec((1, 1, E), lambda b, s: (s[b], 0, 0)),
        ],
        out_specs=pl.BlockSpec((MBLK, E), lambda b, s: (b, 0)),
    )
    ys = pl.pallas_call(
        _gmm_body,
        grid_spec=grid_spec,
        out_shape=jax.ShapeDtypeStruct((PPAD, E), jnp.float32),
        compiler_params=pltpu.CompilerParams(
            dimension_semantics=("arbitrary",),
        ),
    )(bexp.reshape(128), posT, wT, xb, ew1.astype(jnp.bfloat16),
      eb1.reshape(N, 1, H), ew2.astype(jnp.bfloat16), eb2.reshape(N, 1, E))

    combine_sc = pl.kernel(
        _combine_sc_body,
        out_type=jax.ShapeDtypeStruct((S, E), jnp.float32),
        mesh=_SC_MESH,
        scratch_types=[
            pltpu.VMEM((TOK_R,), jnp.int32),
            pltpu.VMEM((TOK_R,), jnp.int32),
            pltpu.VMEM((TOK_R // 2, E), jnp.float32),
            pltpu.VMEM((TOK_R // 2, E), jnp.float32),
            pltpu.SemaphoreType.DMA,
        ],
        compiler_params=_SC_PARAMS,
    )
    out = combine_sc(ys, posT)
    return out.reshape(1, S, E)


# speedup vs baseline: 1.5879x; 1.0377x over previous
import functools

import jax
import jax.numpy as jnp
from jax import lax
from jax.experimental import pallas as pl
from jax.experimental.pallas import tpu as pltpu
from jax.experimental.pallas import tpu_sc as plsc

S = 2048
E = 1024
N = 8
H = 4096
K = 2
P = S * K
MBLK = 128
PPAD = P + N * MBLK
NB = PPAD // MBLK
TBLK = 256
NC = 2
NS = 16
NW = NC * NS
LANES = 16
TOK_R = S // NW

_SC_MESH = plsc.VectorSubcoreMesh(core_axis_name="c", subcore_axis_name="s")
_SC_PARAMS = pltpu.CompilerParams(needs_layout_passes=False)


def _router_body(x_ref, rw1_ref, rb1_ref, rw2_ref, rb2_ref,
                 xbf_ref, wT_ref, posT_ref, bexp_ref,
                 ep_s, rk_s, w_s, carry_ref):
    t = pl.program_id(0)
    nblocks = pl.num_programs(0)
    xbf = x_ref[...].astype(jnp.bfloat16)
    xbf_ref[...] = xbf
    h = jnp.dot(xbf, rw1_ref[...].astype(jnp.bfloat16),
                preferred_element_type=jnp.float32) + rb1_ref[...]
    h = jnp.maximum(h, 0.0).astype(jnp.bfloat16)
    s = jnp.dot(h, rw2_ref[...].astype(jnp.bfloat16),
                preferred_element_type=jnp.float32) + rb2_ref[...]
    lane = lax.broadcasted_iota(jnp.int32, s.shape, 1)
    m1 = jnp.max(s, axis=1, keepdims=True)
    a1 = jnp.min(jnp.where(s == m1, lane, N), axis=1, keepdims=True)
    sm = jnp.where(lane == a1, -jnp.inf, s)
    m2 = jnp.max(sm, axis=1, keepdims=True)
    a2 = jnp.min(jnp.where(sm == m2, lane, N), axis=1, keepdims=True)
    e2 = jnp.exp(m2 - m1)
    w1 = 1.0 / (1.0 + e2)
    w2 = e2 / (1.0 + e2)
    w_s[pl.ds(t * TBLK, TBLK), :] = jnp.concatenate([w1, w2], axis=1)

    oh = (lane == a1).astype(jnp.float32) + (lane == a2).astype(jnp.float32)
    r_i = lax.broadcasted_iota(jnp.int32, (TBLK, TBLK), 0)
    c_i = lax.broadcasted_iota(jnp.int32, (TBLK, TBLK), 1)
    tri = (c_i < r_i).astype(jnp.bfloat16)
    cum = jnp.dot(tri, oh.astype(jnp.bfloat16),
                  preferred_element_type=jnp.float32)

    @pl.when(t == 0)
    def _():
        carry_ref[...] = jnp.zeros_like(carry_ref)

    carry = carry_ref[...]
    cumg = cum + carry
    rank1 = jnp.sum(jnp.where(lane == a1, cumg, 0.0), axis=1, keepdims=True)
    rank2 = jnp.sum(jnp.where(lane == a2, cumg, 0.0), axis=1, keepdims=True)
    ep_s[pl.ds(t * TBLK, TBLK), :] = jnp.concatenate([a1, a2], axis=1)
    rk_s[pl.ds(t * TBLK, TBLK), :] = jnp.concatenate(
        [rank1, rank2], axis=1).astype(jnp.int32)
    carry_ref[...] = carry + jnp.sum(oh, axis=0, keepdims=True)

    @pl.when(t == nblocks - 1)
    def _():
        cnt = carry_ref[...]
        padded = jnp.floor((cnt + (MBLK - 1)) / MBLK) * MBLK
        ui = lax.broadcasted_iota(jnp.int32, (N, N), 0)
        uj = lax.broadcasted_iota(jnp.int32, (N, N), 1)
        offs = jnp.dot(padded.astype(jnp.bfloat16),
                       (ui < uj).astype(jnp.bfloat16),
                       preferred_element_type=jnp.float32)
        offsi = offs + padded
        lane8 = lax.broadcasted_iota(jnp.int32, (1, N), 1)
        ep = ep_s[...]
        rk = rk_s[...].astype(jnp.float32)
        posf = rk
        for e in range(N):
            off_e = jnp.sum(jnp.where(lane8 == e, offs, 0.0))
            posf = posf + jnp.where(ep == e, off_e, 0.0)
        posT_ref[...] = jnp.transpose(posf).astype(jnp.int32)
        wT_ref[...] = jnp.transpose(w_s[...])
        lane128 = lax.broadcasted_iota(jnp.int32, (1, 128), 1)
        row = (lane128 * MBLK).astype(jnp.float32)
        acc = jnp.zeros((1, 128), jnp.int32)
        for e in range(N):
            offi_e = jnp.sum(jnp.where(lane8 == e, offsi, 0.0))
            acc = acc + jnp.where(row >= offi_e, 1, 0)
        bexp_ref[...] = jnp.minimum(acc, N - 1).reshape(128)


def _gmm_body(bexp_ref, posT_ref, wT_ref, xb_ref,
              ew1_ref, eb1_ref, ew2_ref, eb2_ref, o_ref):
    b = pl.program_id(0)
    slot = lax.broadcasted_iota(jnp.int32, (MBLK, S), 0) + b * MBLK
    p0 = posT_ref[0:1, :]
    p1 = posT_ref[1:2, :]
    sel0 = slot == p0
    sel1 = slot == p1
    perm = (sel0 | sel1).astype(jnp.bfloat16)
    xs = jnp.dot(perm, xb_ref[...],
                 preferred_element_type=jnp.float32).astype(jnp.bfloat16)
    ws = jnp.sum(jnp.where(sel0, wT_ref[0:1, :], 0.0) +
                 jnp.where(sel1, wT_ref[1:2, :], 0.0),
                 axis=1, keepdims=True)
    h = jnp.dot(xs, ew1_ref[0], preferred_element_type=jnp.float32) + eb1_ref[0]
    h = jnp.maximum(h, 0.0).astype(jnp.bfloat16)
    y = jnp.dot(h, ew2_ref[0], preferred_element_type=jnp.float32) + eb2_ref[0]
    o_ref[...] = y * ws


def _combine_sc_body(ys_hbm, posT_hbm, out_hbm, idx0_v, idx1_v, rows_a,
                     rows_b, sem):
    cid = lax.axis_index("c")
    sid = lax.axis_index("s")
    wid = sid * NC + cid
    tbase = wid * TOK_R
    pltpu.sync_copy(posT_hbm.at[0, pl.ds(tbase, TOK_R)], idx0_v)
    pltpu.sync_copy(posT_hbm.at[1, pl.ds(tbase, TOK_R)], idx1_v)
    half = TOK_R // 2
    for c in range(2):
        cp1 = pltpu.async_copy(
            ys_hbm.at[idx0_v.at[pl.ds(c * half, half)]], rows_a, sem)
        cp2 = pltpu.async_copy(
            ys_hbm.at[idx1_v.at[pl.ds(c * half, half)]], rows_b, sem)
        cp1.wait()
        cp2.wait()

        @pl.loop(0, half)
        def _(u):
            @pl.loop(0, E, step=LANES)
            def _(j):
                rows_a[u, pl.ds(j, LANES)] = (rows_a[u, pl.ds(j, LANES)] +
                                              rows_b[u, pl.ds(j, LANES)])

        pltpu.sync_copy(rows_a, out_hbm.at[pl.ds(tbase + c * half, half)])


@jax.jit
def kernel(inputs, rw1, rb1, rw2, rb2, ew1, eb1, ew2, eb2):
    x2 = inputs.reshape(S, E)

    xb, wT, posT, bexp = pl.pallas_call(
        _router_body,
        grid=(S // TBLK,),
        in_specs=[
            pl.BlockSpec((TBLK, E), lambda i: (i, 0)),
            pl.BlockSpec((E, E), lambda i: (0, 0)),
            pl.BlockSpec((E,), lambda i: (0,)),
            pl.BlockSpec((E, N), lambda i: (0, 0)),
            pl.BlockSpec((N,), lambda i: (0,)),
        ],
        out_specs=[
            pl.BlockSpec((TBLK, E), lambda i: (i, 0)),
            pl.BlockSpec((K, S), lambda i: (0, 0)),
            pl.BlockSpec((K, S), lambda i: (0, 0)),
            pl.BlockSpec((128,), lambda i: (0,)),
        ],
        out_shape=[
            jax.ShapeDtypeStruct((S, E), jnp.bfloat16),
            jax.ShapeDtypeStruct((K, S), jnp.float32),
            jax.ShapeDtypeStruct((K, S), jnp.int32),
            jax.ShapeDtypeStruct((128,), jnp.int32),
        ],
        scratch_shapes=[
            pltpu.VMEM((S, K), jnp.int32),
            pltpu.VMEM((S, K), jnp.int32),
            pltpu.VMEM((S, K), jnp.float32),
            pltpu.VMEM((1, N), jnp.float32),
        ],
    )(x2, rw1, rb1, rw2, rb2)

    grid_spec = pltpu.PrefetchScalarGridSpec(
        num_scalar_prefetch=1,
        grid=(NB,),
        in_specs=[
            pl.BlockSpec((K, S), lambda b, s: (0, 0)),
            pl.BlockSpec((K, S), lambda b, s: (0, 0)),
            pl.BlockSpec((S, E), lambda b, s: (0, 0)),
            pl.BlockSpec((1, E, H), lambda b, s: (s[b], 0, 0)),
            pl.BlockSpec((1, 1, H), lambda b, s: (s[b], 0, 0)),
            pl.BlockSpec((1, H, E), lambda b, s: (s[b], 0, 0)),
            pl.BlockSpec((1, 1, E), lambda b, s: (s[b], 0, 0)),
        ],
        out_specs=pl.BlockSpec((MBLK, E), lambda b, s: (b, 0)),
    )
    ys = pl.pallas_call(
        _gmm_body,
        grid_spec=grid_spec,
        out_shape=jax.ShapeDtypeStruct((PPAD, E), jnp.float32),
        compiler_params=pltpu.CompilerParams(
            dimension_semantics=("arbitrary",),
        ),
    )(bexp, posT, wT, xb, ew1.astype(jnp.bfloat16),
      eb1.reshape(N, 1, H), ew2.astype(jnp.bfloat16), eb2.reshape(N, 1, E))

    combine_sc = pl.kernel(
        _combine_sc_body,
        out_type=jax.ShapeDtypeStruct((S, E), jnp.float32),
        mesh=_SC_MESH,
        scratch_types=[
            pltpu.VMEM((TOK_R,), jnp.int32),
            pltpu.VMEM((TOK_R,), jnp.int32),
            pltpu.VMEM((TOK_R // 2, E), jnp.float32),
            pltpu.VMEM((TOK_R // 2, E), jnp.float32),
            pltpu.SemaphoreType.DMA,
        ],
        compiler_params=_SC_PARAMS,
    )
    out = combine_sc(ys, posT)
    return out.reshape(1, S, E)
